# baseline (speedup 1.0000x reference)
import functools

import jax
import jax.numpy as jnp
from jax import lax
from jax.experimental import pallas as pl
from jax.experimental.pallas import tpu as pltpu
from jax.experimental.pallas import tpu_sc as plsc

_D = 128
_CH = 128


def _sc_degrees(srcp, dstp, *, nc, ns, npad, cpw):
    mesh = plsc.VectorSubcoreMesh(core_axis_name="c", subcore_axis_name="s")
    nw = nc * ns

    @functools.partial(
        pl.kernel,
        out_type=jax.ShapeDtypeStruct((2 * nw * npad,), jnp.float32),
        mesh=mesh,
        scratch_types=[
            pltpu.VMEM((cpw, _CH), jnp.int32),
            pltpu.VMEM((cpw, _CH), jnp.int32),
            pltpu.VMEM((npad,), jnp.float32),
            pltpu.VMEM((npad,), jnp.float32),
        ],
        compiler_params=pltpu.CompilerParams(needs_layout_passes=False),
    )
    def deg_kernel(srcp_hbm, dstp_hbm, out_hbm, src_v, dst_v, dsrc_l, ddst_l):
        c = lax.axis_index("c")
        s = lax.axis_index("s")
        wid = s * nc + c
        pltpu.sync_copy(srcp_hbm.at[pl.ds(wid * cpw, cpw)], src_v)
        pltpu.sync_copy(dstp_hbm.at[pl.ds(wid * cpw, cpw)], dst_v)
        z = jnp.zeros((16,), jnp.float32)

        def zero(i, carry):
            dsrc_l[pl.ds(i * 16, 16)] = z
            ddst_l[pl.ds(i * 16, 16)] = z
            return carry

        lax.fori_loop(0, npad // 16, zero, 0)
        ones = jnp.ones((16,), jnp.float32)

        def step(j, carry):
            for k in range(_CH // 16):
                idx = src_v[j, pl.ds(k * 16, 16)]
                plsc.addupdate_scatter(dsrc_l, [idx], ones)
            for k in range(_CH // 16):
                idx = dst_v[j, pl.ds(k * 16, 16)]
                plsc.addupdate_scatter(ddst_l, [idx], ones)
            return carry

        lax.fori_loop(0, cpw, step, 0)
        pltpu.sync_copy(dsrc_l, out_hbm.at[pl.ds(wid * npad, npad)])
        pltpu.sync_copy(ddst_l, out_hbm.at[pl.ds((nw + wid) * npad, npad)])

    return deg_kernel(srcp, dstp)


def _sc_aggregate(feat, srcp, dstp, zrows, *, nc, ns, npad, cpw, rps,
                  cpw0, cpw1):
    mesh = plsc.VectorSubcoreMesh(core_axis_name="c", subcore_axis_name="s")
    base1 = ns * cpw0

    @functools.partial(
        pl.kernel,
        out_type=jax.ShapeDtypeStruct((nc, npad, _D), jnp.float32),
        mesh=mesh,
        scratch_types=[
            pltpu.VMEM_SHARED((npad, _D), jnp.float32),
            pltpu.VMEM((cpw0, _CH), jnp.int32),
            pltpu.VMEM((cpw0, _CH), jnp.int32),
            pltpu.VMEM((_CH, _D), jnp.float32),
        ],
    )
    def agg_kernel(feat_hbm, srcp_hbm, dstp_hbm, zr_hbm, out_hbm,
                   agg_sh, src_v, dst_v, buf):
        c = lax.axis_index("c")
        s = lax.axis_index("s")
        off = jnp.where(c == 0, s * cpw0, base1 + s * cpw1)
        my = jnp.where(c == 0, cpw0, cpw1)
        pltpu.sync_copy(zr_hbm, agg_sh.at[pl.ds(s * rps, rps)])
        pltpu.sync_copy(srcp_hbm.at[pl.ds(off, cpw0)], src_v)
        pltpu.sync_copy(dstp_hbm.at[pl.ds(off, cpw0)], dst_v)
        plsc.subcore_barrier()

        def step(j, carry):
            pltpu.sync_copy(feat_hbm.at[src_v.at[j]], buf)
            pltpu.sync_copy(buf, agg_sh.at[dst_v.at[j]], add=True)
            return carry

        lax.fori_loop(0, my, step, 0)
        plsc.subcore_barrier()
        pltpu.sync_copy(agg_sh.at[pl.ds(s * rps, rps)],
                        out_hbm.at[c, pl.ds(s * rps, rps)])

    return agg_kernel(feat, srcp, dstp, zrows)


def _tc_feat(xpad, degs_t, *, npad, nw):
    blk = 256

    def body(x_ref, deg_ref, o_ref):
        d = jnp.sum(deg_ref[0], axis=1, keepdims=True)
        norm = lax.rsqrt(jnp.maximum(d, 1.0))
        o_ref[...] = x_ref[...] * norm

    return pl.pallas_call(
        body,
        grid=(npad // blk,),
        in_specs=[
            pl.BlockSpec((blk, _D), lambda i: (i, 0)),
            pl.BlockSpec((1, blk, nw), lambda i: (0, i, 0)),
        ],
        out_specs=pl.BlockSpec((blk, _D), lambda i: (i, 0)),
        out_shape=jax.ShapeDtypeStruct((npad, _D), jnp.float32),
    )(xpad, degs_t)


def _tc_finish(parts, degs_t, W, b2, *, npad, nw):
    blk = 256

    def body(p_ref, deg_ref, w_ref, b_ref, o_ref):
        agg = p_ref[0] + p_ref[1]
        d = jnp.sum(deg_ref[0], axis=1, keepdims=True)
        norm = lax.rsqrt(jnp.maximum(d, 1.0))
        agg = agg * norm
        o_ref[...] = jnp.dot(agg, w_ref[...],
                             preferred_element_type=jnp.float32) + b_ref[...]

    return pl.pallas_call(
        body,
        grid=(npad // blk,),
        in_specs=[
            pl.BlockSpec((2, blk, _D), lambda i: (0, i, 0)),
            pl.BlockSpec((1, blk, nw), lambda i: (1, i, 0)),
            pl.BlockSpec((_D, _D), lambda i: (0, 0)),
            pl.BlockSpec((1, _D), lambda i: (0, 0)),
        ],
        out_specs=pl.BlockSpec((blk, _D), lambda i: (i, 0)),
        out_shape=jax.ShapeDtypeStruct((npad, _D), jnp.float32),
    )(parts, degs_t, W, b2)


def kernel(x, edge_index, W, b):
    n, d = x.shape
    e = edge_index.shape[1]
    info = plsc.get_sparse_core_info()
    nc, ns = info.num_cores, info.num_subcores
    nw = nc * ns

    rps = -(-(n + 1) // (ns * 128)) * 128
    npad = ns * rps
    cpw = -(-(-(-e // (nw * _CH))) // 8) * 8
    epad = nw * cpw * _CH

    cpw0 = ((2 * cpw * 3) // 4) // 8 * 8
    cpw1 = 2 * cpw - cpw0

    xpad = jnp.zeros((npad, d), jnp.float32).at[:n].set(x)
    fill = jnp.full((epad - e,), n, dtype=jnp.int32)
    srcp = jnp.concatenate([edge_index[0].astype(jnp.int32), fill])
    dstp = jnp.concatenate([edge_index[1].astype(jnp.int32), fill])
    srcp = srcp.reshape(nw * cpw, _CH)
    dstp = dstp.reshape(nw * cpw, _CH)
    slack = jnp.full((cpw0, _CH), n, dtype=jnp.int32)
    srcp_s = jnp.concatenate([srcp, slack])
    dstp_s = jnp.concatenate([dstp, slack])
    zrows = jnp.zeros((rps, d), jnp.float32)

    degs = _sc_degrees(srcp, dstp, nc=nc, ns=ns, npad=npad, cpw=cpw)
    degs_t = degs.reshape(2, nw, npad).transpose(0, 2, 1)
    feat = _tc_feat(xpad, degs_t, npad=npad, nw=nw)
    parts = _sc_aggregate(feat, srcp_s, dstp_s, zrows,
                          nc=nc, ns=ns, npad=npad, cpw=cpw, rps=rps,
                          cpw0=cpw0, cpw1=cpw1)
    out = _tc_finish(parts, degs_t, W, b.reshape(1, d), npad=npad, nw=nw)
    return out[:n]


# speedup vs baseline: 1.0244x; 1.0244x over previous
import functools

import jax
import jax.numpy as jnp
from jax import lax
from jax.experimental import pallas as pl
from jax.experimental.pallas import tpu as pltpu
from jax.experimental.pallas import tpu_sc as plsc

_D = 128
_CH = 128


def _sc_degrees(srcp, dstp, *, nc, ns, npad, cpw):
    mesh = plsc.VectorSubcoreMesh(core_axis_name="c", subcore_axis_name="s")
    nw = nc * ns

    @functools.partial(
        pl.kernel,
        out_type=jax.ShapeDtypeStruct((2 * nw * npad,), jnp.float32),
        mesh=mesh,
        scratch_types=[
            pltpu.VMEM((cpw, _CH), jnp.int32),
            pltpu.VMEM((cpw, _CH), jnp.int32),
            pltpu.VMEM((npad,), jnp.float32),
            pltpu.VMEM((npad,), jnp.float32),
        ],
        compiler_params=pltpu.CompilerParams(needs_layout_passes=False),
    )
    def deg_kernel(srcp_hbm, dstp_hbm, out_hbm, src_v, dst_v, dsrc_l, ddst_l):
        c = lax.axis_index("c")
        s = lax.axis_index("s")
        wid = s * nc + c
        pltpu.sync_copy(srcp_hbm.at[pl.ds(wid * cpw, cpw)], src_v)
        pltpu.sync_copy(dstp_hbm.at[pl.ds(wid * cpw, cpw)], dst_v)
        z = jnp.zeros((16,), jnp.float32)

        def zero(i, carry):
            dsrc_l[pl.ds(i * 16, 16)] = z
            ddst_l[pl.ds(i * 16, 16)] = z
            return carry

        lax.fori_loop(0, npad // 16, zero, 0)
        ones = jnp.ones((16,), jnp.float32)

        def step(j, carry):
            for k in range(_CH // 16):
                idx = src_v[j, pl.ds(k * 16, 16)]
                plsc.addupdate_scatter(dsrc_l, [idx], ones)
            for k in range(_CH // 16):
                idx = dst_v[j, pl.ds(k * 16, 16)]
                plsc.addupdate_scatter(ddst_l, [idx], ones)
            return carry

        lax.fori_loop(0, cpw, step, 0)
        pltpu.sync_copy(dsrc_l, out_hbm.at[pl.ds(wid * npad, npad)])
        pltpu.sync_copy(ddst_l, out_hbm.at[pl.ds((nw + wid) * npad, npad)])

    return deg_kernel(srcp, dstp)


def _sc_aggregate(feat, srcp, dstp, zrows, *, nc, ns, npad, cpw, rps,
                  cpw0, cpw1):
    mesh = plsc.VectorSubcoreMesh(core_axis_name="c", subcore_axis_name="s")
    base1 = ns * cpw0

    @functools.partial(
        pl.kernel,
        out_type=jax.ShapeDtypeStruct((nc, npad, _D), jnp.float32),
        mesh=mesh,
        scratch_types=[
            pltpu.VMEM_SHARED((npad, _D), jnp.float32),
            pltpu.VMEM((cpw0, _CH), jnp.int32),
            pltpu.VMEM((cpw0, _CH), jnp.int32),
            pltpu.VMEM((_CH, _D), jnp.float32),
        ],
    )
    def agg_kernel(feat_hbm, srcp_hbm, dstp_hbm, zr_hbm, out_hbm,
                   agg_sh, src_v, dst_v, buf):
        c = lax.axis_index("c")
        s = lax.axis_index("s")
        off = jnp.where(c == 0, s * cpw0, base1 + s * cpw1)
        my = jnp.where(c == 0, cpw0, cpw1)
        pltpu.sync_copy(zr_hbm, agg_sh.at[pl.ds(s * rps, rps)])
        pltpu.sync_copy(srcp_hbm.at[pl.ds(off, cpw0)], src_v)
        pltpu.sync_copy(dstp_hbm.at[pl.ds(off, cpw0)], dst_v)
        plsc.subcore_barrier()

        def step(j, carry):
            pltpu.sync_copy(feat_hbm.at[src_v.at[j]], buf)
            pltpu.sync_copy(buf, agg_sh.at[dst_v.at[j]], add=True)
            return carry

        lax.fori_loop(0, my, step, 0)
        plsc.subcore_barrier()
        pltpu.sync_copy(agg_sh.at[pl.ds(s * rps, rps)],
                        out_hbm.at[c, pl.ds(s * rps, rps)])

    return agg_kernel(feat, srcp, dstp, zrows)


def _tc_feat(xpad, degs_t, *, npad, nw):
    blk = 256

    def body(x_ref, deg_ref, o_ref):
        d = jnp.sum(deg_ref[0], axis=1, keepdims=True)
        norm = lax.rsqrt(jnp.maximum(d, 1.0))
        o_ref[...] = x_ref[...] * norm

    return pl.pallas_call(
        body,
        grid=(npad // blk,),
        in_specs=[
            pl.BlockSpec((blk, _D), lambda i: (i, 0)),
            pl.BlockSpec((1, blk, nw), lambda i: (0, i, 0)),
        ],
        out_specs=pl.BlockSpec((blk, _D), lambda i: (i, 0)),
        out_shape=jax.ShapeDtypeStruct((npad, _D), jnp.float32),
    )(xpad, degs_t)


def _tc_finish(parts, degs_t, W, b2, *, npad, nw):
    blk = 256

    def body(p_ref, deg_ref, w_ref, b_ref, o_ref):
        agg = p_ref[0] + p_ref[1]
        d = jnp.sum(deg_ref[0], axis=1, keepdims=True)
        norm = lax.rsqrt(jnp.maximum(d, 1.0))
        agg = agg * norm
        o_ref[...] = jnp.dot(agg, w_ref[...],
                             preferred_element_type=jnp.float32) + b_ref[...]

    return pl.pallas_call(
        body,
        grid=(npad // blk,),
        in_specs=[
            pl.BlockSpec((2, blk, _D), lambda i: (0, i, 0)),
            pl.BlockSpec((1, blk, nw), lambda i: (1, i, 0)),
            pl.BlockSpec((_D, _D), lambda i: (0, 0)),
            pl.BlockSpec((1, _D), lambda i: (0, 0)),
        ],
        out_specs=pl.BlockSpec((blk, _D), lambda i: (i, 0)),
        out_shape=jax.ShapeDtypeStruct((npad, _D), jnp.float32),
    )(parts, degs_t, W, b2)


def kernel(x, edge_index, W, b):
    n, d = x.shape
    e = edge_index.shape[1]
    info = plsc.get_sparse_core_info()
    nc, ns = info.num_cores, info.num_subcores
    nw = nc * ns

    rps = -(-(n + 1) // (ns * 128)) * 128
    npad = ns * rps
    cpw = -(-(-(-e // (nw * _CH))) // 8) * 8
    epad = nw * cpw * _CH

    cpw0 = ((2 * cpw * 4) // 5) // 8 * 8
    cpw1 = 2 * cpw - cpw0

    xpad = jnp.zeros((npad, d), jnp.float32).at[:n].set(x)
    fill = jnp.full((epad - e,), n, dtype=jnp.int32)
    srcp = jnp.concatenate([edge_index[0].astype(jnp.int32), fill])
    dstp = jnp.concatenate([edge_index[1].astype(jnp.int32), fill])
    srcp = srcp.reshape(nw * cpw, _CH)
    dstp = dstp.reshape(nw * cpw, _CH)
    slack = jnp.full((cpw0, _CH), n, dtype=jnp.int32)
    srcp_s = jnp.concatenate([srcp, slack])
    dstp_s = jnp.concatenate([dstp, slack])
    zrows = jnp.zeros((rps, d), jnp.float32)

    degs = _sc_degrees(srcp, dstp, nc=nc, ns=ns, npad=npad, cpw=cpw)
    degs_t = degs.reshape(2, nw, npad).transpose(0, 2, 1)
    feat = _tc_feat(xpad, degs_t, npad=npad, nw=nw)
    parts = _sc_aggregate(feat, srcp_s, dstp_s, zrows,
                          nc=nc, ns=ns, npad=npad, cpw=cpw, rps=rps,
                          cpw0=cpw0, cpw1=cpw1)
    out = _tc_finish(parts, degs_t, W, b.reshape(1, d), npad=npad, nw=nw)
    return out[:n]
